# baseline (speedup 1.0000x reference)
import functools

import jax
import jax.numpy as jnp
from jax import lax
from jax.experimental import pallas as pl
from jax.experimental.pallas import tpu as pltpu
from jax.experimental.pallas import tpu_sc as plsc

VOCAB = 100000
DIM = 128
BATCH = 4096
HIST = 50

_info = plsc.get_sparse_core_info()
_NC, _NS = _info.num_cores, _info.num_subcores
NW = _NC * _NS
TOTAL = BATCH * HIST
PER_W = TOTAL // NW
CHUNK = 64
PAIR = 2 * CHUNK
NPAIR = PER_W // PAIR
NBUF = 5
NGROUP = NPAIR // NBUF


def _emb_body(table, idx, out, idx_v, *rest):
    bufs = rest[:NBUF]
    gsems = rest[NBUF:2 * NBUF]
    osems = rest[2 * NBUF:3 * NBUF]
    wid = lax.axis_index("s") * _NC + lax.axis_index("c")
    base = wid * PER_W
    pltpu.sync_copy(idx.at[pl.ds(base, PER_W)], idx_v)

    def gather(p, b):
        pltpu.async_copy(table.at[idx_v.at[pl.ds(p * PAIR, CHUNK)]],
                         bufs[b].at[pl.ds(0, CHUNK)], gsems[b])
        pltpu.async_copy(table.at[idx_v.at[pl.ds(p * PAIR + CHUNK, CHUNK)]],
                         bufs[b].at[pl.ds(CHUNK, CHUNK)], gsems[b])

    for b in range(NBUF):
        gather(b, b)

    def group(g, carry):
        for b in range(NBUF):
            p = g * NBUF + b
            pltpu.make_async_copy(table.at[idx_v.at[pl.ds(0, PAIR)]],
                                  bufs[b], gsems[b]).wait()
            pltpu.async_copy(bufs[b], out.at[pl.ds(base + p * PAIR, PAIR)],
                             osems[b])

        @pl.when(g < NGROUP - 1)
        def _():
            for b in range(NBUF):
                pltpu.make_async_copy(bufs[b], out.at[pl.ds(0, PAIR)],
                                      osems[b]).wait()
                gather((g + 1) * NBUF + b, b)
        return carry

    lax.fori_loop(0, NGROUP, group, 0)
    for b in range(NBUF):
        pltpu.make_async_copy(bufs[b], out.at[pl.ds(0, PAIR)], osems[b]).wait()


_emb_call = functools.partial(
    pl.kernel,
    out_type=jax.ShapeDtypeStruct((TOTAL, DIM), jnp.float32),
    mesh=plsc.VectorSubcoreMesh(core_axis_name="c", subcore_axis_name="s"),
    scratch_types=(
        [pltpu.VMEM((PER_W,), jnp.int32)]
        + [pltpu.VMEM((PAIR, DIM), jnp.float32) for _ in range(NBUF)]
        + [pltpu.SemaphoreType.DMA for _ in range(2 * NBUF)]
    ),
)(_emb_body)


def kernel(inputs, embeddings):
    idx = inputs.astype(jnp.int32).T.reshape(TOTAL)
    out = _emb_call(embeddings, idx)
    return out.reshape(HIST, BATCH, DIM).transpose(1, 0, 2)


# speedup vs baseline: 1.0124x; 1.0124x over previous
import functools

import jax
import jax.numpy as jnp
from jax import lax
from jax.experimental import pallas as pl
from jax.experimental.pallas import tpu as pltpu
from jax.experimental.pallas import tpu_sc as plsc

VOCAB = 100000
DIM = 128
BATCH = 4096
HIST = 50

_info = plsc.get_sparse_core_info()
_NC, _NS = _info.num_cores, _info.num_subcores
NW = _NC * _NS
TOTAL = BATCH * HIST
PER_W = TOTAL // NW
CHUNK = 64
NCHUNK = PER_W // CHUNK
NBUF = 10
NGROUP = NCHUNK // NBUF


def _emb_body(table, idx, out, idx_v, *rest):
    bufs = rest[:NBUF]
    gsems = rest[NBUF:2 * NBUF]
    osems = rest[2 * NBUF:3 * NBUF]
    wid = lax.axis_index("s") * _NC + lax.axis_index("c")
    base = wid * PER_W
    pltpu.sync_copy(idx.at[pl.ds(base, PER_W)], idx_v)

    def gather(c, b):
        pltpu.async_copy(table.at[idx_v.at[pl.ds(c * CHUNK, CHUNK)]],
                         bufs[b], gsems[b])

    for b in range(NBUF):
        gather(b, b)

    def group(g, carry):
        for b in range(NBUF):
            c = g * NBUF + b
            pltpu.make_async_copy(table.at[idx_v.at[pl.ds(0, CHUNK)]],
                                  bufs[b], gsems[b]).wait()
            pltpu.async_copy(bufs[b], out.at[pl.ds(base + c * CHUNK, CHUNK)],
                             osems[b])

        @pl.when(g < NGROUP - 1)
        def _():
            for b in range(NBUF):
                pltpu.make_async_copy(bufs[b], out.at[pl.ds(0, CHUNK)],
                                      osems[b]).wait()
                gather((g + 1) * NBUF + b, b)
        return carry

    lax.fori_loop(0, NGROUP, group, 0)
    for b in range(NBUF):
        pltpu.make_async_copy(bufs[b], out.at[pl.ds(0, CHUNK)], osems[b]).wait()


_emb_call = functools.partial(
    pl.kernel,
    out_type=jax.ShapeDtypeStruct((TOTAL, DIM), jnp.float32),
    mesh=plsc.VectorSubcoreMesh(core_axis_name="c", subcore_axis_name="s"),
    scratch_types=(
        [pltpu.VMEM((PER_W,), jnp.int32)]
        + [pltpu.VMEM((CHUNK, DIM), jnp.float32) for _ in range(NBUF)]
        + [pltpu.SemaphoreType.DMA for _ in range(2 * NBUF)]
    ),
)(_emb_body)


def kernel(inputs, embeddings):
    idx = inputs.astype(jnp.int32).T.reshape(TOTAL)
    out = _emb_call(embeddings, idx)
    return out.reshape(HIST, BATCH, DIM).transpose(1, 0, 2)
